# baseline (speedup 1.0000x reference)
import functools

import jax
import jax.numpy as jnp
from jax import lax
from jax.experimental import pallas as pl
from jax.experimental.pallas import tpu as pltpu
from jax.experimental.pallas import tpu_sc as plsc

N = 10000
D = 128
E = 320000

NC = 2
NS = 16
NW = NC * NS
CH = 128
NCH = 80
EPWP = NCH * CH
EP = NW * EPWP
ACC = 10112
RPT = ACC // NS
YR = N + 8

_HI = lax.Precision.HIGHEST

_sc_mesh = plsc.VectorSubcoreMesh(core_axis_name="c", subcore_axis_name="s")


def _doubling_t(taus_row, s, n):
    ii = lax.broadcasted_iota(jnp.int32, (n, n), 0)
    jj = lax.broadcasted_iota(jnp.int32, (n, n), 1)
    t = jnp.where(ii == jj, jnp.broadcast_to(taus_row, (n, n)), 0.0)
    lvl = 1
    while lvl < n:
        m = (((ii // lvl) % 2) == 0) & (((jj // lvl) % 2) == 1) \
            & ((ii // (2 * lvl)) == (jj // (2 * lvl)))
        sm = jnp.where(m, s, 0.0)
        tsm = lax.dot_general(t, sm, (((1,), (0,)), ((), ())), precision=_HI)
        t = t - lax.dot_general(tsm, t, (((1,), (0,)), ((), ())), precision=_HI)
        lvl *= 2
    return t


def _unpack_chunk(pk_v, j, si_v, di_v):
    for c in range(CH // 16):
        w = pk_v[j, pl.ds(c * 16, 16)]
        si_v[pl.ds(c * 16, 16)] = lax.bitwise_and(
            w, jnp.full((16,), 0xFFFF, jnp.int32))
        di_v[pl.ds(c * 16, 16)] = lax.shift_right_logical(
            w, jnp.full((16,), 16, jnp.int32))


@functools.partial(
    pl.kernel,
    out_type=jax.ShapeDtypeStruct((NW, ACC), jnp.float32),
    mesh=_sc_mesh,
    compiler_params=pltpu.CompilerParams(needs_layout_passes=False),
    scratch_types=[
        pltpu.VMEM((EPWP,), jnp.int32),
        pltpu.VMEM((ACC,), jnp.float32),
    ],
)
def _k1_deg(pk_hbm, out_hbm, pk_v, hist_v):
    cid = lax.axis_index("c")
    sid = lax.axis_index("s")
    wid = cid * NS + sid

    for i in range(ACC // 16):
        hist_v[pl.ds(i * 16, 16)] = jnp.zeros((16,), jnp.float32)

    pltpu.sync_copy(pk_hbm.at[wid], pk_v)

    ones16 = jnp.ones((16,), jnp.float32)
    m16 = jnp.full((16,), 0xFFFF, jnp.int32)
    for t in range(EPWP // 16):
        w = pk_v[pl.ds(t * 16, 16)]
        plsc.addupdate_scatter(hist_v, [lax.bitwise_and(w, m16)], ones16)

    pltpu.sync_copy(hist_v, out_hbm.at[wid])


@functools.partial(
    pl.kernel,
    out_type=jax.ShapeDtypeStruct((NC, ACC, D), jnp.float32),
    mesh=_sc_mesh,
    scratch_types=[
        pltpu.VMEM((NCH, CH), jnp.int32),
        pltpu.VMEM((CH,), jnp.int32),
        pltpu.VMEM((CH,), jnp.int32),
        pltpu.VMEM((CH,), jnp.int32),
        pltpu.VMEM((CH,), jnp.int32),
        pltpu.VMEM((CH, D), jnp.float32),
        pltpu.VMEM((CH, D), jnp.float32),
        pltpu.VMEM_SHARED((ACC, D), jnp.float32),
        pltpu.SemaphoreType.DMA,
        pltpu.SemaphoreType.DMA,
    ],
)
def _k3_segsum(y_hbm, pk_hbm, out_hbm, pk_v, si0, di0, si1, di1,
               buf0, buf1, acc_sh, sem0, sem1):
    cid = lax.axis_index("c")
    sid = lax.axis_index("s")
    wid = cid * NS + sid

    for cb in range(D // 16):
        def zfill(r, c, _cb=cb):
            buf0[r, pl.ds(_cb * 16, 16)] = jnp.zeros((16,), jnp.float32)
            return c

        lax.fori_loop(0, CH, zfill, 0)

    base = sid * RPT
    for m in range(RPT // CH):
        pltpu.sync_copy(buf0, acc_sh.at[pl.ds(base + m * CH, CH)])
    rem = RPT - (RPT // CH) * CH
    if rem:
        pltpu.sync_copy(buf0.at[pl.ds(0, rem)],
                        acc_sh.at[pl.ds(base + (RPT // CH) * CH, rem)])
    plsc.subcore_barrier()

    pltpu.sync_copy(pk_hbm.at[wid], pk_v)

    _unpack_chunk(pk_v, 0, si0, di0)
    pltpu.async_copy(y_hbm.at[di0], buf0, sem0)

    def pair(t, c):
        j = 2 * t
        _unpack_chunk(pk_v, j + 1, si1, di1)
        pltpu.make_async_copy(y_hbm.at[di0], buf0, sem0).wait()
        pltpu.async_copy(y_hbm.at[di1], buf1, sem1)
        pltpu.sync_copy(buf0, acc_sh.at[si0], add=True)
        jn = lax.rem(j + 2, NCH)
        _unpack_chunk(pk_v, jn, si0, di0)
        pltpu.make_async_copy(y_hbm.at[di1], buf1, sem1).wait()
        pltpu.async_copy(y_hbm.at[di0], buf0, sem0)
        pltpu.sync_copy(buf1, acc_sh.at[si1], add=True)
        return c

    lax.fori_loop(0, NCH // 2, pair, 0)
    pltpu.make_async_copy(y_hbm.at[di0], buf0, sem0).wait()
    plsc.subcore_barrier()

    pltpu.sync_copy(acc_sh.at[pl.ds(base, RPT)],
                    out_hbm.at[cid, pl.ds(base, RPT)])


def _k2a_body(dp_ref, out_ref):
    out_ref[...] = jnp.sum(dp_ref[...], axis=0, keepdims=True)


def _k2a_reduce(degp):
    return pl.pallas_call(
        _k2a_body,
        in_specs=[pl.BlockSpec(memory_space=pltpu.VMEM)],
        out_specs=pl.BlockSpec(memory_space=pltpu.VMEM),
        out_shape=jax.ShapeDtypeStruct((1, ACC), jnp.float32),
    )(degp)


_RB2 = 1000


def _k2_body(x_ref, d_ref, y_ref):
    y_ref[...] = x_ref[...] * lax.rsqrt(d_ref[...])


def _k2_scale(x, dc):
    return pl.pallas_call(
        _k2_body,
        grid=(N // _RB2,),
        in_specs=[
            pl.BlockSpec((_RB2, D), lambda i: (i, 0)),
            pl.BlockSpec((_RB2, 1), lambda i: (i, 0)),
        ],
        out_specs=pl.BlockSpec((_RB2, D), lambda i: (i, 0)),
        out_shape=jax.ShapeDtypeStruct((N, D), jnp.float32),
    )(x, dc)


_PW = 32
_NP = D // _PW


def _k4_body(xt_ref, s0_ref, s1_ref, d_ref, vt_ref, u_ref,
             at_ref, taus_ref):
    deg = d_ref[...]
    dis = jnp.where(deg > 0.0, lax.rsqrt(deg), 0.0)
    at_ref[...] = (0.5 * dis) * (s0_ref[...] + s1_ref[...]) \
        + (0.5 * deg) * xt_ref[...]
    taus_ref[...] = jnp.zeros((1, D), jnp.float32)

    lane = lax.broadcasted_iota(jnp.int32, (1, N), 1)
    laned = lax.broadcasted_iota(jnp.int32, (1, D), 1)

    for p in range(_NP):
        lo, hi = p * _PW, (p + 1) * _PW

        def step(k, carry, _lo=lo, _hi=hi):
            row = at_ref[pl.ds(k, 1), :]
            alpha = jnp.sum(jnp.where(lane == k, row, 0.0),
                            axis=(0, 1), keepdims=True)
            rowm = jnp.where(lane > k, row, 0.0)
            sigma = jnp.sum(rowm * rowm, axis=(0, 1), keepdims=True)
            mu = jnp.sqrt(alpha * alpha + sigma)
            sig0 = sigma == 0.0
            beta = jnp.where(sig0, alpha, jnp.where(alpha < 0.0, mu, -mu))
            tau = jnp.where(sig0, 0.0, (beta - alpha) / beta)
            dvs = jnp.where(sig0, 1.0, alpha - beta)
            v = jnp.where(lane == k, 1.0, rowm / dvs)
            vt_ref[pl.ds(k, 1), :] = v
            taus_ref[...] = jnp.where(laned == k, tau, taus_ref[...])
            a = at_ref[_lo:_hi, :]
            w = jnp.sum(a * v, axis=1, keepdims=True)
            at_ref[_lo:_hi, :] = a - (tau * w) * v
            return carry

        lax.fori_loop(lo, hi, step, 0)

        if p < _NP - 1:
            vp = vt_ref[lo:hi, :]
            sp_ = lax.dot_general(vp, vp, (((1,), (1,)), ((), ())),
                                  precision=_HI)
            tp = _doubling_t(taus_ref[0:1, lo:hi], sp_, _PW)
            ar = at_ref[hi:D, :]
            cc = lax.dot_general(vp, ar, (((1,), (1,)), ((), ())),
                                 precision=_HI)
            d2 = lax.dot_general(tp, cc, (((0,), (0,)), ((), ())),
                                 precision=_HI)
            at_ref[hi:D, :] = ar - lax.dot_general(
                d2, vp, (((0,), (0,)), ((), ())), precision=_HI)

    vt = vt_ref[...]
    s = lax.dot_general(vt, vt, (((1,), (1,)), ((), ())), precision=_HI)
    t = _doubling_t(taus_ref[...], s, D)
    u_ref[...] = lax.dot_general(t, vt, (((1,), (0,)), ((), ())), precision=_HI)


def _k4_qr(xt, s0t, s1t, dr):
    return pl.pallas_call(
        _k4_body,
        in_specs=[pl.BlockSpec(memory_space=pltpu.VMEM)] * 4,
        out_specs=[pl.BlockSpec(memory_space=pltpu.VMEM)] * 2,
        out_shape=[
            jax.ShapeDtypeStruct((D, N), jnp.float32),
            jax.ShapeDtypeStruct((D, N), jnp.float32),
        ],
        scratch_shapes=[
            pltpu.VMEM((D, N), jnp.float32),
            pltpu.VMEM((1, D), jnp.float32),
        ],
    )(xt, s0t, s1t, dr)


_RB = 400


def _k5_body(x0_ref, v_ref, u_ref, run_ref, out_ref):
    i = pl.program_id(0)
    q = -lax.dot_general(v_ref[...], u_ref[...], (((1,), (0,)), ((), ())),
                         precision=lax.Precision.DEFAULT)
    r = lax.broadcasted_iota(jnp.int32, (_RB, N), 0) + i * _RB
    c = lax.broadcasted_iota(jnp.int32, (_RB, N), 1)
    q = q + (r == c).astype(jnp.float32)
    runv = run_ref[0, 0]
    out_ref[0, :, 0:D] = x0_ref[0, :, :] * runv
    out_ref[0, :, D:] = q * runv


def _k5_out(x0, v, u, runf):
    return pl.pallas_call(
        _k5_body,
        grid=(N // _RB,),
        in_specs=[
            pl.BlockSpec((1, _RB, D), lambda i: (0, i, 0)),
            pl.BlockSpec((_RB, D), lambda i: (i, 0)),
            pl.BlockSpec((D, N), lambda i: (0, 0)),
            pl.BlockSpec((1, 1), lambda i: (0, 0)),
        ],
        out_specs=pl.BlockSpec((1, _RB, D + N), lambda i: (0, i, 0)),
        out_shape=jax.ShapeDtypeStruct((1, N, D + N), jnp.float32),
        compiler_params=pltpu.CompilerParams(
            dimension_semantics=("arbitrary",)),
    )(x0, v, u, runf)


def kernel(x0, edge_index, run=1):
    x = x0[0]
    src = edge_index[:, 0]
    dst = edge_index[:, 1]
    padi = N + (lax.iota(jnp.int32, EP - E) % 8)
    pk = jnp.bitwise_or(jnp.concatenate([src, padi]),
                        jnp.concatenate([dst, padi]) << 16)
    pkp = pk.reshape(NW, NCH, CH)

    degp = _k1_deg(pk.reshape(NW, EPWP))
    degrow = _k2a_reduce(degp)
    dcol = degrow.reshape(ACC)[:N, None]

    y = _k2_scale(x, dcol)
    ypad = jnp.concatenate([y, jnp.zeros((YR - N, D), jnp.float32)], axis=0)

    sp = _k3_segsum(ypad, pkp)

    vt, u = _k4_qr(x.T, sp[0, :N].T, sp[1, :N].T, degrow[:, :N])

    runf = jnp.asarray(run, x0.dtype).reshape(1, 1)
    return _k5_out(x0, vt.T, u, runf)


# speedup vs baseline: 1.0777x; 1.0777x over previous
import functools

import jax
import jax.numpy as jnp
from jax import lax
from jax.experimental import pallas as pl
from jax.experimental.pallas import tpu as pltpu
from jax.experimental.pallas import tpu_sc as plsc

N = 10000
D = 128
E = 320000

NC = 2
NS = 16
NW = NC * NS
CH = 128
NCH = 80
EPWP = NCH * CH
EP = NW * EPWP
ACC = 10112
RPT = ACC // NS
YR = N + 8

_HI = lax.Precision.DEFAULT

_sc_mesh = plsc.VectorSubcoreMesh(core_axis_name="c", subcore_axis_name="s")


def _doubling_t(taus_row, s, n):
    ii = lax.broadcasted_iota(jnp.int32, (n, n), 0)
    jj = lax.broadcasted_iota(jnp.int32, (n, n), 1)
    t = jnp.where(ii == jj, jnp.broadcast_to(taus_row, (n, n)), 0.0)
    lvl = 1
    while lvl < n:
        m = (((ii // lvl) % 2) == 0) & (((jj // lvl) % 2) == 1) \
            & ((ii // (2 * lvl)) == (jj // (2 * lvl)))
        sm = jnp.where(m, s, 0.0)
        tsm = lax.dot_general(t, sm, (((1,), (0,)), ((), ())), precision=_HI)
        t = t - lax.dot_general(tsm, t, (((1,), (0,)), ((), ())), precision=_HI)
        lvl *= 2
    return t


def _unpack_chunk(pk_v, j, si_v, di_v):
    for c in range(CH // 16):
        w = pk_v[j, pl.ds(c * 16, 16)]
        si_v[pl.ds(c * 16, 16)] = lax.bitwise_and(
            w, jnp.full((16,), 0xFFFF, jnp.int32))
        di_v[pl.ds(c * 16, 16)] = lax.shift_right_logical(
            w, jnp.full((16,), 16, jnp.int32))


@functools.partial(
    pl.kernel,
    out_type=jax.ShapeDtypeStruct((NW, ACC), jnp.float32),
    mesh=_sc_mesh,
    compiler_params=pltpu.CompilerParams(needs_layout_passes=False),
    scratch_types=[
        pltpu.VMEM((EPWP,), jnp.int32),
        pltpu.VMEM((ACC,), jnp.float32),
    ],
)
def _k1_deg(pk_hbm, out_hbm, pk_v, hist_v):
    cid = lax.axis_index("c")
    sid = lax.axis_index("s")
    wid = cid * NS + sid

    for i in range(ACC // 16):
        hist_v[pl.ds(i * 16, 16)] = jnp.zeros((16,), jnp.float32)

    pltpu.sync_copy(pk_hbm.at[wid], pk_v)

    ones16 = jnp.ones((16,), jnp.float32)
    m16 = jnp.full((16,), 0xFFFF, jnp.int32)
    for t in range(EPWP // 16):
        w = pk_v[pl.ds(t * 16, 16)]
        plsc.addupdate_scatter(hist_v, [lax.bitwise_and(w, m16)], ones16)

    pltpu.sync_copy(hist_v, out_hbm.at[wid])


@functools.partial(
    pl.kernel,
    out_type=jax.ShapeDtypeStruct((NC, ACC, D), jnp.float32),
    mesh=_sc_mesh,
    scratch_types=[
        pltpu.VMEM((NCH, CH), jnp.int32),
        pltpu.VMEM((CH,), jnp.int32),
        pltpu.VMEM((CH,), jnp.int32),
        pltpu.VMEM((CH,), jnp.int32),
        pltpu.VMEM((CH,), jnp.int32),
        pltpu.VMEM((CH, D), jnp.float32),
        pltpu.VMEM((CH, D), jnp.float32),
        pltpu.VMEM_SHARED((ACC, D), jnp.float32),
        pltpu.SemaphoreType.DMA,
        pltpu.SemaphoreType.DMA,
    ],
)
def _k3_segsum(y_hbm, pk_hbm, out_hbm, pk_v, si0, di0, si1, di1,
               buf0, buf1, acc_sh, sem0, sem1):
    cid = lax.axis_index("c")
    sid = lax.axis_index("s")
    wid = cid * NS + sid

    for cb in range(D // 16):
        def zfill(r, c, _cb=cb):
            buf0[r, pl.ds(_cb * 16, 16)] = jnp.zeros((16,), jnp.float32)
            return c

        lax.fori_loop(0, CH, zfill, 0)

    base = sid * RPT
    for m in range(RPT // CH):
        pltpu.sync_copy(buf0, acc_sh.at[pl.ds(base + m * CH, CH)])
    rem = RPT - (RPT // CH) * CH
    if rem:
        pltpu.sync_copy(buf0.at[pl.ds(0, rem)],
                        acc_sh.at[pl.ds(base + (RPT // CH) * CH, rem)])
    plsc.subcore_barrier()

    pltpu.sync_copy(pk_hbm.at[wid], pk_v)

    _unpack_chunk(pk_v, 0, si0, di0)
    pltpu.async_copy(y_hbm.at[di0], buf0, sem0)

    def pair(t, c):
        j = 2 * t
        _unpack_chunk(pk_v, j + 1, si1, di1)
        pltpu.make_async_copy(y_hbm.at[di0], buf0, sem0).wait()
        pltpu.async_copy(y_hbm.at[di1], buf1, sem1)
        pltpu.sync_copy(buf0, acc_sh.at[si0], add=True)
        jn = lax.rem(j + 2, NCH)
        _unpack_chunk(pk_v, jn, si0, di0)
        pltpu.make_async_copy(y_hbm.at[di1], buf1, sem1).wait()
        pltpu.async_copy(y_hbm.at[di0], buf0, sem0)
        pltpu.sync_copy(buf1, acc_sh.at[si1], add=True)
        return c

    lax.fori_loop(0, NCH // 2, pair, 0)
    pltpu.make_async_copy(y_hbm.at[di0], buf0, sem0).wait()
    plsc.subcore_barrier()

    pltpu.sync_copy(acc_sh.at[pl.ds(base, RPT)],
                    out_hbm.at[cid, pl.ds(base, RPT)])


def _k2a_body(dp_ref, out_ref):
    out_ref[...] = jnp.sum(dp_ref[...], axis=0, keepdims=True)


def _k2a_reduce(degp):
    return pl.pallas_call(
        _k2a_body,
        in_specs=[pl.BlockSpec(memory_space=pltpu.VMEM)],
        out_specs=pl.BlockSpec(memory_space=pltpu.VMEM),
        out_shape=jax.ShapeDtypeStruct((1, ACC), jnp.float32),
    )(degp)


_RB2 = 1000


def _k2_body(x_ref, d_ref, y_ref):
    y_ref[...] = x_ref[...] * lax.rsqrt(d_ref[...])


def _k2_scale(x, dc):
    return pl.pallas_call(
        _k2_body,
        grid=(N // _RB2,),
        in_specs=[
            pl.BlockSpec((_RB2, D), lambda i: (i, 0)),
            pl.BlockSpec((_RB2, 1), lambda i: (i, 0)),
        ],
        out_specs=pl.BlockSpec((_RB2, D), lambda i: (i, 0)),
        out_shape=jax.ShapeDtypeStruct((N, D), jnp.float32),
    )(x, dc)


_PW = 32
_NP = D // _PW


def _k4_body(xt_ref, s0_ref, s1_ref, d_ref, vt_ref, u_ref,
             at_ref, taus_ref):
    deg = d_ref[...]
    dis = jnp.where(deg > 0.0, lax.rsqrt(deg), 0.0)
    at_ref[...] = (0.5 * dis) * (s0_ref[...] + s1_ref[...]) \
        + (0.5 * deg) * xt_ref[...]
    taus_ref[...] = jnp.zeros((1, D), jnp.float32)

    lane = lax.broadcasted_iota(jnp.int32, (1, N), 1)
    laned = lax.broadcasted_iota(jnp.int32, (1, D), 1)

    for p in range(_NP):
        lo, hi = p * _PW, (p + 1) * _PW

        def step(k, carry, _lo=lo, _hi=hi):
            row = at_ref[pl.ds(k, 1), :]
            alpha = jnp.sum(jnp.where(lane == k, row, 0.0),
                            axis=(0, 1), keepdims=True)
            rowm = jnp.where(lane > k, row, 0.0)
            sigma = jnp.sum(rowm * rowm, axis=(0, 1), keepdims=True)
            mu = jnp.sqrt(alpha * alpha + sigma)
            sig0 = sigma == 0.0
            beta = jnp.where(sig0, alpha, jnp.where(alpha < 0.0, mu, -mu))
            tau = jnp.where(sig0, 0.0, (beta - alpha) / beta)
            dvs = jnp.where(sig0, 1.0, alpha - beta)
            v = jnp.where(lane == k, 1.0, rowm / dvs)
            vt_ref[pl.ds(k, 1), :] = v
            taus_ref[...] = jnp.where(laned == k, tau, taus_ref[...])
            a = at_ref[_lo:_hi, :]
            w = jnp.sum(a * v, axis=1, keepdims=True)
            at_ref[_lo:_hi, :] = a - (tau * w) * v
            return carry

        lax.fori_loop(lo, hi, step, 0)

        if p < _NP - 1:
            vp = vt_ref[lo:hi, :]
            sp_ = lax.dot_general(vp, vp, (((1,), (1,)), ((), ())),
                                  precision=_HI)
            tp = _doubling_t(taus_ref[0:1, lo:hi], sp_, _PW)
            ar = at_ref[hi:D, :]
            cc = lax.dot_general(vp, ar, (((1,), (1,)), ((), ())),
                                 precision=_HI)
            d2 = lax.dot_general(tp, cc, (((0,), (0,)), ((), ())),
                                 precision=_HI)
            at_ref[hi:D, :] = ar - lax.dot_general(
                d2, vp, (((0,), (0,)), ((), ())), precision=_HI)

    vt = vt_ref[...]
    s = lax.dot_general(vt, vt, (((1,), (1,)), ((), ())), precision=_HI)
    t = _doubling_t(taus_ref[...], s, D)
    u_ref[...] = lax.dot_general(t, vt, (((1,), (0,)), ((), ())), precision=_HI)


def _k4_qr(xt, s0t, s1t, dr):
    return pl.pallas_call(
        _k4_body,
        in_specs=[pl.BlockSpec(memory_space=pltpu.VMEM)] * 4,
        out_specs=[pl.BlockSpec(memory_space=pltpu.VMEM)] * 2,
        out_shape=[
            jax.ShapeDtypeStruct((D, N), jnp.float32),
            jax.ShapeDtypeStruct((D, N), jnp.float32),
        ],
        scratch_shapes=[
            pltpu.VMEM((D, N), jnp.float32),
            pltpu.VMEM((1, D), jnp.float32),
        ],
    )(xt, s0t, s1t, dr)


_RB = 400


def _k5_body(x0_ref, v_ref, u_ref, run_ref, out_ref):
    i = pl.program_id(0)
    q = -lax.dot_general(v_ref[...], u_ref[...], (((1,), (0,)), ((), ())),
                         precision=lax.Precision.DEFAULT)
    r = lax.broadcasted_iota(jnp.int32, (_RB, N), 0) + i * _RB
    c = lax.broadcasted_iota(jnp.int32, (_RB, N), 1)
    q = q + (r == c).astype(jnp.float32)
    runv = run_ref[0, 0]
    out_ref[0, :, 0:D] = x0_ref[0, :, :] * runv
    out_ref[0, :, D:] = q * runv


def _k5_out(x0, v, u, runf):
    return pl.pallas_call(
        _k5_body,
        grid=(N // _RB,),
        in_specs=[
            pl.BlockSpec((1, _RB, D), lambda i: (0, i, 0)),
            pl.BlockSpec((_RB, D), lambda i: (i, 0)),
            pl.BlockSpec((D, N), lambda i: (0, 0)),
            pl.BlockSpec((1, 1), lambda i: (0, 0)),
        ],
        out_specs=pl.BlockSpec((1, _RB, D + N), lambda i: (0, i, 0)),
        out_shape=jax.ShapeDtypeStruct((1, N, D + N), jnp.float32),
        compiler_params=pltpu.CompilerParams(
            dimension_semantics=("arbitrary",)),
    )(x0, v, u, runf)


def kernel(x0, edge_index, run=1):
    x = x0[0]
    src = edge_index[:, 0]
    dst = edge_index[:, 1]
    padi = N + (lax.iota(jnp.int32, EP - E) % 8)
    pk = jnp.bitwise_or(jnp.concatenate([src, padi]),
                        jnp.concatenate([dst, padi]) << 16)
    pkp = pk.reshape(NW, NCH, CH)

    degp = _k1_deg(pk.reshape(NW, EPWP))
    degrow = _k2a_reduce(degp)
    dcol = degrow.reshape(ACC)[:N, None]

    y = _k2_scale(x, dcol)
    ypad = jnp.concatenate([y, jnp.zeros((YR - N, D), jnp.float32)], axis=0)

    sp = _k3_segsum(ypad, pkp)

    vt, u = _k4_qr(x.T, sp[0, :N].T, sp[1, :N].T, degrow[:, :N])

    runf = jnp.asarray(run, x0.dtype).reshape(1, 1)
    return _k5_out(x0, vt.T, u, runf)
